# baseline (speedup 1.0000x reference)
import jax
import jax.numpy as jnp
from jax import lax
from jax.experimental import pallas as pl
from jax.experimental.pallas import tpu as pltpu
from jax.experimental.pallas import tpu_sc as plsc

FEATURE_DIM = 100000
EMBEDDING_DIM = 64

NUM_CORES = 2
NUM_SUBCORES = 16
NUM_WORKERS = NUM_CORES * NUM_SUBCORES

BATCH = 4096
SEQ = 50
TOTAL = BATCH * SEQ
PER_WORKER = TOTAL // NUM_WORKERS
B_W = BATCH // NUM_WORKERS
LANES = 16


def _iota16():
    return jax.lax.iota(jnp.int32, 16)


def _gather_body(idx_hbm, table_hbm, out_hbm,
                 idx_v, idx_s, rows0, rows1, stag0, stag1,
                 g0, g1, s0, s1):
    wid = lax.axis_index("s") * NUM_CORES + lax.axis_index("c")
    base = wid * PER_WORKER
    b0 = wid * B_W

    rows_b = (rows0, rows1)
    stag_b = (stag0, stag1)
    gsem = (g0, g1)
    ssem = (s0, s1)

    pltpu.sync_copy(idx_hbm.at[pl.ds(base, PER_WORKER)], idx_v)

    row_ids = [_iota16() + LANES * kb for kb in range(B_W // LANES)]

    @pl.loop(0, SEQ)
    def _(s):
        for kb in range(B_W // LANES):
            ids = row_ids[kb] * SEQ + s
            v = plsc.load_gather(idx_v, [ids])
            idx_s[s, pl.ds(kb * LANES, LANES)] = v

    def issue_gather(s, p):
        return pltpu.async_copy(table_hbm.at[idx_s.at[s]], rows_b[p], gsem[p])

    def drain_gather(p):
        pltpu.make_async_copy(table_hbm.at[pl.ds(0, B_W)], rows_b[p],
                              gsem[p]).wait()

    def issue_store(s, p):
        return pltpu.async_copy(stag_b[p],
                                out_hbm.at[s, :, pl.ds(b0, B_W)], ssem[p])

    def drain_store(p):
        pltpu.make_async_copy(stag_b[p],
                              out_hbm.at[0, :, pl.ds(b0, B_W)],
                              ssem[p]).wait()

    def transpose(p):
        rows = rows_b[p]
        stag = stag_b[p]

        @pl.loop(0, EMBEDDING_DIM, unroll=4)
        def _(c):
            col = jnp.zeros((LANES,), jnp.int32) + c
            for kb in range(B_W // LANES):
                v = plsc.load_gather(rows, [row_ids[kb], col])
                stag[c, pl.ds(kb * LANES, LANES)] = v

    issue_gather(0, 0)
    issue_gather(1, 1)

    drain_gather(0)
    transpose(0)
    issue_gather(2, 0)
    issue_store(0, 0)
    drain_gather(1)
    transpose(1)
    issue_gather(3, 1)
    issue_store(1, 1)

    @pl.loop(1, SEQ // 2 - 1)
    def _(t):
        s = 2 * t
        for p, si in ((0, s), (1, s + 1)):
            drain_gather(p)
            drain_store(p)
            transpose(p)
            issue_gather(si + 2, p)
            issue_store(si, p)

    for p, si in ((0, SEQ - 2), (1, SEQ - 1)):
        drain_gather(p)
        drain_store(p)
        transpose(p)
        issue_store(si, p)
    drain_store(0)
    drain_store(1)


@jax.jit
def _gather(idx, table):
    mesh = plsc.VectorSubcoreMesh(core_axis_name="c", subcore_axis_name="s",
                                  num_cores=NUM_CORES,
                                  num_subcores=NUM_SUBCORES)
    return pl.kernel(
        _gather_body,
        out_type=jax.ShapeDtypeStruct((SEQ, EMBEDDING_DIM, BATCH),
                                      jnp.float32),
        mesh=mesh,
        scratch_types=[
            pltpu.VMEM((PER_WORKER,), jnp.int32),
            pltpu.VMEM((SEQ, B_W), jnp.int32),
            pltpu.VMEM((B_W, EMBEDDING_DIM), jnp.float32),
            pltpu.VMEM((B_W, EMBEDDING_DIM), jnp.float32),
            pltpu.VMEM((EMBEDDING_DIM, B_W), jnp.float32),
            pltpu.VMEM((EMBEDDING_DIM, B_W), jnp.float32),
            pltpu.SemaphoreType.DMA,
            pltpu.SemaphoreType.DMA,
            pltpu.SemaphoreType.DMA,
            pltpu.SemaphoreType.DMA,
        ],
        compiler_params=pltpu.CompilerParams(use_tc_tiling_on_sc=False,
                                             needs_layout_passes=False),
    )(idx, table)


def kernel(x, table):
    idx = x.reshape(-1).astype(jnp.int32)
    out3 = _gather(idx, table)
    return out3.transpose(2, 0, 1)


# speedup vs baseline: 1.7931x; 1.7931x over previous
import jax
import jax.numpy as jnp
from jax import lax
from jax.experimental import pallas as pl
from jax.experimental.pallas import tpu as pltpu
from jax.experimental.pallas import tpu_sc as plsc

FEATURE_DIM = 100000
EMBEDDING_DIM = 64

NUM_CORES = 2
NUM_SUBCORES = 16
NUM_WORKERS = NUM_CORES * NUM_SUBCORES

BATCH = 4096
SEQ = 50
TOTAL = BATCH * SEQ
PER_WORKER = TOTAL // NUM_WORKERS
CHUNK = 800
NUM_CHUNKS = PER_WORKER // CHUNK


def _gather_body(idx_hbm, table_hbm, out_hbm,
                 idx_v, rows0, rows1, g0, g1, s0, s1):
    wid = lax.axis_index("s") * NUM_CORES + lax.axis_index("c")
    base = wid * PER_WORKER

    rows_b = (rows0, rows1)
    gsem = (g0, g1)
    ssem = (s0, s1)
    gathers = [None, None]
    stores = [None, None]

    pltpu.sync_copy(idx_hbm.at[pl.ds(base, PER_WORKER)], idx_v)

    for i in range(NUM_CHUNKS):
        b = i % 2
        if i >= 2:
            stores[b].wait()
        gathers[b] = pltpu.async_copy(
            table_hbm.at[idx_v.at[pl.ds(i * CHUNK, CHUNK)]], rows_b[b],
            gsem[b])
        if i >= 1:
            pb = (i - 1) % 2
            gathers[pb].wait()
            stores[pb] = pltpu.async_copy(
                rows_b[pb],
                out_hbm.at[pl.ds(base + (i - 1) * CHUNK, CHUNK)],
                ssem[pb])

    last = (NUM_CHUNKS - 1) % 2
    gathers[last].wait()
    stores[last] = pltpu.async_copy(
        rows_b[last],
        out_hbm.at[pl.ds(base + (NUM_CHUNKS - 1) * CHUNK, CHUNK)],
        ssem[last])
    stores[1 - last].wait()
    stores[last].wait()


@jax.jit
def _gather(idx, table):
    mesh = plsc.VectorSubcoreMesh(core_axis_name="c", subcore_axis_name="s",
                                  num_cores=NUM_CORES,
                                  num_subcores=NUM_SUBCORES)
    return pl.kernel(
        _gather_body,
        out_type=jax.ShapeDtypeStruct((TOTAL, EMBEDDING_DIM), jnp.float32),
        mesh=mesh,
        scratch_types=[
            pltpu.VMEM((PER_WORKER,), jnp.int32),
            pltpu.VMEM((CHUNK, EMBEDDING_DIM), jnp.float32),
            pltpu.VMEM((CHUNK, EMBEDDING_DIM), jnp.float32),
            pltpu.SemaphoreType.DMA,
            pltpu.SemaphoreType.DMA,
            pltpu.SemaphoreType.DMA,
            pltpu.SemaphoreType.DMA,
        ],
        compiler_params=pltpu.CompilerParams(use_tc_tiling_on_sc=False),
    )(idx, table)


def kernel(x, table):
    idx = x.reshape(-1).astype(jnp.int32)
    out2d = _gather(idx, table)
    return out2d.reshape(BATCH, SEQ, EMBEDDING_DIM)
